# baseline (speedup 1.0000x reference)
import functools

import jax
import jax.numpy as jnp
from jax import lax
from jax.experimental import pallas as pl
from jax.experimental.pallas import tpu as pltpu
from jax.experimental.pallas import tpu_sc as plsc

_B = 4096 * 200
_D = 64
_NC = 2
_NS = 16
_NW = _NC * _NS
_BPW = _B // _NW
_CHUNK = 512
_NCHUNK = _BPW // _CHUNK


def _gather_body(idx_hbm, table_hbm, out_hbm, idx_v, rows_v, gsem):
    wid = lax.axis_index("s") * _NC + lax.axis_index("c")
    base = wid * _BPW
    pltpu.sync_copy(idx_hbm.at[pl.ds(base, _BPW)], idx_v)

    def step(c, carry):
        cbase = c * _CHUNK
        pltpu.async_copy(
            table_hbm.at[idx_v.at[pl.ds(cbase, _CHUNK)]], rows_v, gsem
        ).wait()
        pltpu.sync_copy(rows_v, out_hbm.at[pl.ds(base + cbase, _CHUNK)])
        return carry

    lax.fori_loop(0, _NCHUNK, step, 0)


@functools.partial(jax.jit, donate_argnums=())
def _embedding_gather(x_flat, emb_weight):
    mesh = plsc.VectorSubcoreMesh(core_axis_name="c", subcore_axis_name="s")
    run = functools.partial(
        pl.kernel,
        mesh=mesh,
        out_type=jax.ShapeDtypeStruct((_B, _D), jnp.float32),
        scratch_types=[
            pltpu.VMEM((_BPW,), jnp.int32),
            pltpu.VMEM((_CHUNK, _D), jnp.float32),
            pltpu.SemaphoreType.DMA,
        ],
        compiler_params=pltpu.CompilerParams(use_tc_tiling_on_sc=False),
    )(_gather_body)
    return run(x_flat, emb_weight)


def kernel(x, emb_weight):
    out = _embedding_gather(x.reshape(-1).astype(jnp.int32), emb_weight)
    return out.reshape(x.shape + (_D,))


# speedup vs baseline: 1.0282x; 1.0282x over previous
import functools

import jax
import jax.numpy as jnp
from jax import lax
from jax.experimental import pallas as pl
from jax.experimental.pallas import tpu as pltpu
from jax.experimental.pallas import tpu_sc as plsc

_B = 4096 * 200
_D = 64
_NC = 2
_NS = 16
_NW = _NC * _NS
_BPW = _B // _NW
_CHUNK = 512
_NCHUNK = _BPW // _CHUNK


def _gather_body(idx_hbm, table_hbm, out_hbm, idx_v, rows0, rows1,
                 gsem0, gsem1, wsem0, wsem1):
    wid = lax.axis_index("s") * _NC + lax.axis_index("c")
    base = wid * _BPW
    pltpu.sync_copy(idx_hbm.at[pl.ds(base, _BPW)], idx_v)

    bufs = (rows0, rows1)
    gsems = (gsem0, gsem1)
    wsems = (wsem0, wsem1)

    def gather_copy(c, b):
        return pltpu.make_async_copy(
            table_hbm.at[idx_v.at[pl.ds(c * _CHUNK, _CHUNK)]],
            bufs[b], gsems[b],
        )

    def write_copy(c, b):
        return pltpu.make_async_copy(
            bufs[b], out_hbm.at[pl.ds(base + c * _CHUNK, _CHUNK)], wsems[b],
        )

    gather_copy(0, 0).start()
    gather_copy(1, 1).start()
    gather_copy(0, 0).wait()
    write_copy(0, 0).start()

    def step(g, carry):
        for p in range(2):
            c = 1 + g * 2 + p
            b, ob = (1 + p) % 2, p % 2
            write_copy(c - 1, ob).wait()
            gather_copy(c + 1, ob).start()
            gather_copy(c, b).wait()
            write_copy(c, b).start()
        return carry

    lax.fori_loop(0, (_NCHUNK - 2) // 2, step, 0)

    c = _NCHUNK - 1
    gather_copy(c, c % 2).wait()
    write_copy(c, c % 2).start()
    write_copy(c - 1, (c - 1) % 2).wait()
    write_copy(c, c % 2).wait()


@functools.partial(jax.jit, donate_argnums=())
def _embedding_gather(x_flat, emb_weight):
    mesh = plsc.VectorSubcoreMesh(core_axis_name="c", subcore_axis_name="s")
    run = functools.partial(
        pl.kernel,
        mesh=mesh,
        out_type=jax.ShapeDtypeStruct((_B, _D), jnp.float32),
        scratch_types=[
            pltpu.VMEM((_BPW,), jnp.int32),
            pltpu.VMEM((_CHUNK, _D), jnp.float32),
            pltpu.VMEM((_CHUNK, _D), jnp.float32),
            pltpu.SemaphoreType.DMA,
            pltpu.SemaphoreType.DMA,
            pltpu.SemaphoreType.DMA,
            pltpu.SemaphoreType.DMA,
        ],
        compiler_params=pltpu.CompilerParams(use_tc_tiling_on_sc=False),
    )(_gather_body)
    return run(x_flat, emb_weight)


def kernel(x, emb_weight):
    out = _embedding_gather(x.reshape(-1).astype(jnp.int32), emb_weight)
    return out.reshape(x.shape + (_D,))
